# baseline (speedup 1.0000x reference)
import numpy as np

import jax
import jax.numpy as jnp
from jax import lax
from jax.experimental import pallas as pl
from jax.experimental.pallas import tpu as pltpu
from jax.experimental.pallas import tpu_sc as plsc

N = 10000
E = 320000
D = 128
TW = 160
EXT = 136
NC = 2
NS = 16
NW = NC * NS
C = 48
NCH = 210
EPW = NCH * C
EP = NW * EPW
GRP = C // 16
RPT = N // NS
ZR = 25

_PIN = np.empty((D,), np.int64)
for _j in range(4):
    for _i in range(16):
        _PIN[32 * _j + 2 * _i] = 32 * _j + _i
        _PIN[32 * _j + 2 * _i + 1] = 32 * _j + 16 + _i
_PIN = tuple(int(x) for x in _PIN)


def _prep_body(beta_ref, feat_ref, tbl_ref, tbld_ref, z_ref):
    x = feat_ref[...]
    ss = jnp.sum(x * x, axis=1, keepdims=True)
    nmax = jnp.maximum(jnp.sqrt(ss), 1e-12)
    nh = x / nmax
    b = jnp.full((N, 1), beta_ref[0, 0], jnp.float32)
    pad = jnp.zeros((N, TW - D - 4), jnp.float32)
    row = jnp.concatenate([nh, nmax, nmax, b, b, pad], axis=1)
    tbl_ref[...] = row.astype(jnp.bfloat16)
    tbld_ref[...] = nh.astype(jnp.bfloat16)
    z_ref[...] = jnp.zeros((N, EXT), jnp.float32)


def _prep(featp, beta):
    return pl.pallas_call(
        _prep_body,
        in_specs=[
            pl.BlockSpec(memory_space=pltpu.SMEM),
            pl.BlockSpec(memory_space=pltpu.VMEM),
        ],
        out_shape=[
            jax.ShapeDtypeStruct((N, TW), jnp.bfloat16),
            jax.ShapeDtypeStruct((N, D), jnp.bfloat16),
            jax.ShapeDtypeStruct((N, EXT), jnp.float32),
        ],
    )(jnp.reshape(beta, (1, 1)), featp)


def _finalize_body(parts_ref, out_ref):
    ext = parts_ref[0] + parts_ref[1]
    den = jnp.maximum(ext[:, D:D + 1], 1e-30)
    out_ref[...] = ext[:, :D] / den


def _finalize(parts):
    return pl.pallas_call(
        _finalize_body,
        out_shape=jax.ShapeDtypeStruct((N, D), jnp.float32),
    )(parts)


def _sc_body(tbl_hbm, tbld_hbm, sd_hbm, z_hbm, out_hbm, acc_sh,
             idx0, idx1, sx0, sx1, fs0, fs1, fd0, fd1, mg0, mg1,
             srow, is0, is1, gs0, gs1, ss0, ss1):
    cid = lax.axis_index("c")
    sid = lax.axis_index("s")
    wid = cid * NS + sid
    idxs = [idx0, idx1]
    sidx = [sx0, sx1]
    fss = [fs0, fs1]
    fds = [fd0, fd1]
    msgs = [mg0, mg1]
    isems = [is0, is1]
    gsems = [gs0, gs1]
    ssems = [ss0, ss1]

    pltpu.sync_copy(z_hbm.at[pl.ds(sid * RPT, RPT)],
                    acc_sh.at[pl.ds(sid * RPT, RPT)])

    def zmsg(r, _):
        mg0[r, pl.ds(EXT - 16, 16)] = jnp.zeros((16,), jnp.float32)
        mg1[r, pl.ds(EXT - 16, 16)] = jnp.zeros((16,), jnp.float32)
        return 0
    lax.fori_loop(0, C, zmsg, 0)
    plsc.subcore_barrier()

    rows16 = lax.broadcasted_iota(jnp.int32, (16,), 0)
    ebase0 = wid * EPW

    def stage_idx(p, s):
        base = ebase0 + p * C
        pltpu.async_copy(sd_hbm.at[0, pl.ds(base, C)], idxs[s].at[0], isems[s])
        pltpu.async_copy(sd_hbm.at[1, pl.ds(base, C)], idxs[s].at[1], isems[s])

    def issue_gather(p, s):
        base = ebase0 + p * C
        pltpu.make_async_copy(
            sd_hbm.at[0, pl.ds(base, C)], idxs[s].at[0], isems[s]).wait()
        pltpu.make_async_copy(
            sd_hbm.at[1, pl.ds(base, C)], idxs[s].at[1], isems[s]).wait()
        pltpu.async_copy(tbl_hbm.at[idxs[s].at[0]], fss[s], gsems[s])
        pltpu.async_copy(tbld_hbm.at[idxs[s].at[1]], fds[s], gsems[s])

    def wait_gather(s):
        pltpu.make_async_copy(tbl_hbm.at[idxs[s].at[0]], fss[s], gsems[s]).wait()
        pltpu.make_async_copy(tbld_hbm.at[idxs[s].at[1]], fds[s], gsems[s]).wait()

    def issue_scatter(s):
        pltpu.async_copy(msgs[s], acc_sh.at[sidx[s]], ssems[s], add=True)

    def wait_scatter(s):
        pltpu.make_async_copy(msgs[s], acc_sh.at[sidx[s]], ssems[s]).wait()

    def unpack2(v):
        return plsc.unpack(v, format=plsc.PackFormat.INTERLEAVED,
                           preferred_element_type=jnp.float32)

    def compute(p, s):
        fs, fd, msg = fss[s], fds[s], msgs[s]
        bscal = None
        for g in range(GRP):
            e0 = g * 16
            nm = [None] * 16
            for e in range(16):
                row = e0 + e
                dp = None
                for j in range(D // 32):
                    qa, qb = unpack2(fs[row, pl.ds(j * 32, 32)])
                    ta, tb = unpack2(fd[row, pl.ds(j * 32, 32)])
                    term = qa * ta + qb * tb
                    dp = term if dp is None else dp + term
                srow[e, :] = dp
                sa, _sb = unpack2(fs[row, pl.ds(D, 32)])
                nm[e] = sa[0]
                if bscal is None:
                    bscal = sa[1]
            tot = plsc.load_gather(srow, [rows16, jnp.zeros((16,), jnp.int32)])
            for j in range(1, 16):
                tot = tot + plsc.load_gather(
                    srow, [rows16, jnp.full((16,), j, jnp.int32)])
            w = jnp.exp(tot * bscal)
            gidx = ebase0 + p * C + e0 + rows16
            w = jnp.where(gidx < E, w, 0.0)
            for e in range(16):
                row = e0 + e
                ws2 = w[e] * nm[e]
                for j in range(D // 32):
                    qa, qb = unpack2(fs[row, pl.ds(j * 32, 32)])
                    msg[row, pl.ds(j * 32, 16)] = qa * ws2
                    msg[row, pl.ds(j * 32 + 16, 16)] = qb * ws2
            plsc.store_scatter(
                msg, [rows16 + e0, jnp.full((16,), D, jnp.int32)], w)

    stage_idx(0, 0)
    stage_idx(1, 1)
    issue_gather(0, 0)
    issue_gather(1, 1)

    KMAX = NCH // 2

    def body(k, _):
        for r in range(2):
            s = r
            p = 2 * k + r
            with jax.named_scope("wgather"):
                wait_gather(s)
            for j in range(C // 16):
                sidx[s][pl.ds(j * 16, 16)] = idxs[s][1, pl.ds(j * 16, 16)]
            @pl.when(k < KMAX - 1)
            def _():
                stage_idx(p + 2, s)
            with jax.named_scope("comp"):
                compute(p, s)
            with jax.named_scope("wscat"):
                if r == 0:
                    @pl.when(k > 0)
                    def _():
                        wait_scatter(1)
                else:
                    wait_scatter(0)
            issue_scatter(s)
            @pl.when(k < KMAX - 1)
            def _():
                issue_gather(p + 2, s)
        return 0

    lax.fori_loop(0, KMAX, body, 0)
    wait_scatter(1)
    plsc.subcore_barrier()

    pltpu.sync_copy(acc_sh.at[pl.ds(sid * RPT, RPT)],
                    out_hbm.at[cid, pl.ds(sid * RPT, RPT)])


def _sc_edge_pass(tbl, tbld, sd, zeros):
    mesh = plsc.VectorSubcoreMesh(core_axis_name="c", subcore_axis_name="s")
    return pl.kernel(
        _sc_body,
        out_type=jax.ShapeDtypeStruct((NC, N, EXT), jnp.float32),
        mesh=mesh,
        compiler_params=pltpu.CompilerParams(
            use_tc_tiling_on_sc=False, needs_layout_passes=False),
        scratch_types=(
            [pltpu.VMEM_SHARED((N, EXT), jnp.float32)]
            + [pltpu.VMEM((2, C), jnp.int32)] * 2
            + [pltpu.VMEM((C,), jnp.int32)] * 2
            + [pltpu.VMEM((C, TW), jnp.bfloat16)] * 2
            + [pltpu.VMEM((C, D), jnp.bfloat16)] * 2
            + [pltpu.VMEM((C, EXT), jnp.float32)] * 2
            + [pltpu.VMEM((16, 16), jnp.float32)]
            + [pltpu.SemaphoreType.DMA] * 6
        ),
    )(tbl, tbld, sd, zeros)


def kernel(feat, edge_index, beta):
    sd = jnp.pad(edge_index.astype(jnp.int32), ((0, 0), (0, EP - E)))
    tbl, tbld, zeros = _prep(feat[:, list(_PIN)], beta.astype(jnp.float32))
    parts = _sc_edge_pass(tbl, tbld, sd, zeros)
    return _finalize(parts)


# speedup vs baseline: 1.0177x; 1.0177x over previous
import numpy as np

import jax
import jax.numpy as jnp
from jax import lax
from jax.experimental import pallas as pl
from jax.experimental.pallas import tpu as pltpu
from jax.experimental.pallas import tpu_sc as plsc

N = 10000
E = 320000
D = 128
TW = 160
EXT = 144
NC = 2
NS = 16
NW = NC * NS
C = 48
NCH = 210
EPW = NCH * C
EP = NW * EPW
GRP = C // 16
RPT = N // NS
ZR = 25

_PIN = np.empty((D,), np.int64)
for _j in range(4):
    for _i in range(16):
        _PIN[32 * _j + 2 * _i] = 32 * _j + _i
        _PIN[32 * _j + 2 * _i + 1] = 32 * _j + 16 + _i
_PIN = tuple(int(x) for x in _PIN)


def _prep_body(beta_ref, feat_ref, tbl_ref, tbld_ref, z_ref):
    x = feat_ref[...]
    ss = jnp.sum(x * x, axis=1, keepdims=True)
    nmax = jnp.maximum(jnp.sqrt(ss), 1e-12)
    nh = x / nmax
    b = jnp.full((N, 1), beta_ref[0, 0], jnp.float32)
    pad = jnp.zeros((N, TW - D - 4), jnp.float32)
    row = jnp.concatenate([nh, nmax, nmax, b, b, pad], axis=1)
    tbl_ref[...] = row.astype(jnp.bfloat16)
    tbld_ref[...] = nh.astype(jnp.bfloat16)
    z_ref[...] = jnp.zeros((N, EXT), jnp.float32)


def _prep(featp, beta):
    return pl.pallas_call(
        _prep_body,
        in_specs=[
            pl.BlockSpec(memory_space=pltpu.SMEM),
            pl.BlockSpec(memory_space=pltpu.VMEM),
        ],
        out_shape=[
            jax.ShapeDtypeStruct((N, TW), jnp.bfloat16),
            jax.ShapeDtypeStruct((N, D), jnp.bfloat16),
            jax.ShapeDtypeStruct((N, EXT), jnp.float32),
        ],
    )(jnp.reshape(beta, (1, 1)), featp)


def _finalize_body(parts_ref, out_ref):
    ext = parts_ref[0] + parts_ref[1]
    den = jnp.maximum(ext[:, D:D + 1], 1e-30)
    out_ref[...] = ext[:, :D] / den


def _finalize(parts):
    return pl.pallas_call(
        _finalize_body,
        out_shape=jax.ShapeDtypeStruct((N, D), jnp.float32),
    )(parts)


def _sc_body(tbl_hbm, tbld_hbm, sd_hbm, z_hbm, out_hbm, acc_sh,
             idx0, idx1, sx0, sx1, fs0, fs1, fd0, fd1, mg0, mg1,
             srow, is0, is1, gs0, gs1, ss0, ss1):
    cid = lax.axis_index("c")
    sid = lax.axis_index("s")
    wid = cid * NS + sid
    idxs = [idx0, idx1]
    sidx = [sx0, sx1]
    fss = [fs0, fs1]
    fds = [fd0, fd1]
    msgs = [mg0, mg1]
    isems = [is0, is1]
    gsems = [gs0, gs1]
    ssems = [ss0, ss1]

    pltpu.sync_copy(z_hbm.at[pl.ds(sid * RPT, RPT)],
                    acc_sh.at[pl.ds(sid * RPT, RPT)])

    def zmsg(r, _):
        mg0[r, pl.ds(D, 16)] = jnp.zeros((16,), jnp.float32)
        mg1[r, pl.ds(D, 16)] = jnp.zeros((16,), jnp.float32)
        return 0
    lax.fori_loop(0, C, zmsg, 0)
    plsc.subcore_barrier()

    rows16 = lax.broadcasted_iota(jnp.int32, (16,), 0)
    ebase0 = wid * EPW

    def stage_idx(p, s):
        base = ebase0 + p * C
        pltpu.async_copy(sd_hbm.at[0, pl.ds(base, C)], idxs[s].at[0], isems[s])
        pltpu.async_copy(sd_hbm.at[1, pl.ds(base, C)], idxs[s].at[1], isems[s])

    def issue_gather(p, s):
        base = ebase0 + p * C
        pltpu.make_async_copy(
            sd_hbm.at[0, pl.ds(base, C)], idxs[s].at[0], isems[s]).wait()
        pltpu.make_async_copy(
            sd_hbm.at[1, pl.ds(base, C)], idxs[s].at[1], isems[s]).wait()
        pltpu.async_copy(tbl_hbm.at[idxs[s].at[0]], fss[s], gsems[s])
        pltpu.async_copy(tbld_hbm.at[idxs[s].at[1]], fds[s], gsems[s])

    def wait_gather(s):
        pltpu.make_async_copy(tbl_hbm.at[idxs[s].at[0]], fss[s], gsems[s]).wait()
        pltpu.make_async_copy(tbld_hbm.at[idxs[s].at[1]], fds[s], gsems[s]).wait()

    def issue_scatter(s):
        pltpu.async_copy(msgs[s], acc_sh.at[sidx[s]], ssems[s], add=True)

    def wait_scatter(s):
        pltpu.make_async_copy(msgs[s], acc_sh.at[sidx[s]], ssems[s]).wait()

    def unpack2(v):
        return plsc.unpack(v, format=plsc.PackFormat.INTERLEAVED,
                           preferred_element_type=jnp.float32)

    def compute(p, s):
        fs, fd, msg = fss[s], fds[s], msgs[s]
        bscal = None
        for g in range(GRP):
            e0 = g * 16
            nm = [None] * 16
            for e in range(16):
                row = e0 + e
                dp = None
                for j in range(D // 32):
                    qa, qb = unpack2(fs[row, pl.ds(j * 32, 32)])
                    ta, tb = unpack2(fd[row, pl.ds(j * 32, 32)])
                    term = qa * ta + qb * tb
                    dp = term if dp is None else dp + term
                srow[e, :] = dp
                sa, _sb = unpack2(fs[row, pl.ds(D, 32)])
                nm[e] = sa[0]
                if bscal is None:
                    bscal = sa[1]
            tot = plsc.load_gather(srow, [rows16, jnp.zeros((16,), jnp.int32)])
            for j in range(1, 16):
                tot = tot + plsc.load_gather(
                    srow, [rows16, jnp.full((16,), j, jnp.int32)])
            w = jnp.exp(tot * bscal)
            gidx = ebase0 + p * C + e0 + rows16
            w = jnp.where(gidx < E, w, 0.0)
            for e in range(16):
                row = e0 + e
                ws2 = w[e] * nm[e]
                for j in range(D // 32):
                    qa, qb = unpack2(fs[row, pl.ds(j * 32, 32)])
                    msg[row, pl.ds(j * 32, 16)] = qa * ws2
                    msg[row, pl.ds(j * 32 + 16, 16)] = qb * ws2
            plsc.store_scatter(
                msg, [rows16 + e0, jnp.full((16,), D, jnp.int32)], w)

    stage_idx(0, 0)
    stage_idx(1, 1)
    issue_gather(0, 0)
    issue_gather(1, 1)

    KMAX = NCH // 2

    def body(k, _):
        for r in range(2):
            s = r
            p = 2 * k + r
            with jax.named_scope("wgather"):
                wait_gather(s)
            for j in range(C // 16):
                sidx[s][pl.ds(j * 16, 16)] = idxs[s][1, pl.ds(j * 16, 16)]
            @pl.when(k < KMAX - 1)
            def _():
                stage_idx(p + 2, s)
            with jax.named_scope("comp"):
                compute(p, s)
            with jax.named_scope("wscat"):
                if r == 0:
                    @pl.when(k > 0)
                    def _():
                        wait_scatter(1)
                else:
                    wait_scatter(0)
            issue_scatter(s)
            @pl.when(k < KMAX - 1)
            def _():
                issue_gather(p + 2, s)
        return 0

    lax.fori_loop(0, KMAX, body, 0)
    wait_scatter(1)
    plsc.subcore_barrier()

    pltpu.sync_copy(acc_sh.at[pl.ds(sid * RPT, RPT)],
                    out_hbm.at[cid, pl.ds(sid * RPT, RPT)])


def _sc_edge_pass(tbl, tbld, sd, zeros):
    mesh = plsc.VectorSubcoreMesh(core_axis_name="c", subcore_axis_name="s")
    return pl.kernel(
        _sc_body,
        out_type=jax.ShapeDtypeStruct((NC, N, EXT), jnp.float32),
        mesh=mesh,
        compiler_params=pltpu.CompilerParams(
            use_tc_tiling_on_sc=False, needs_layout_passes=False),
        scratch_types=(
            [pltpu.VMEM_SHARED((N, EXT), jnp.float32)]
            + [pltpu.VMEM((2, C), jnp.int32)] * 2
            + [pltpu.VMEM((C,), jnp.int32)] * 2
            + [pltpu.VMEM((C, TW), jnp.bfloat16)] * 2
            + [pltpu.VMEM((C, D), jnp.bfloat16)] * 2
            + [pltpu.VMEM((C, EXT), jnp.float32)] * 2
            + [pltpu.VMEM((16, 16), jnp.float32)]
            + [pltpu.SemaphoreType.DMA] * 6
        ),
    )(tbl, tbld, sd, zeros)


def kernel(feat, edge_index, beta):
    sd = jnp.pad(edge_index.astype(jnp.int32), ((0, 0), (0, EP - E)))
    tbl, tbld, zeros = _prep(feat[:, list(_PIN)], beta.astype(jnp.float32))
    parts = _sc_edge_pass(tbl, tbld, sd, zeros)
    return _finalize(parts)


# speedup vs baseline: 1.0667x; 1.0482x over previous
import numpy as np

import jax
import jax.numpy as jnp
from jax import lax
from jax.experimental import pallas as pl
from jax.experimental.pallas import tpu as pltpu
from jax.experimental.pallas import tpu_sc as plsc

N = 10000
E = 320000
D = 128
TW = 160
EXT = 144
NC = 2
NS = 16
NW = NC * NS
C = 48
NCH = 210
EPW = NCH * C
EP = NW * EPW
GRP = C // 16
RPT = N // NS
ZR = 25

_PIN = np.empty((D,), np.int64)
for _j in range(4):
    for _i in range(16):
        _PIN[32 * _j + 2 * _i] = 32 * _j + _i
        _PIN[32 * _j + 2 * _i + 1] = 32 * _j + 16 + _i
_PIN = tuple(int(x) for x in _PIN)


def _prep_body(beta_ref, feat_ref, tbl_ref, tbld_ref):
    x = feat_ref[...]
    ss = jnp.sum(x * x, axis=1, keepdims=True)
    nmax = jnp.maximum(jnp.sqrt(ss), 1e-12)
    nh = x / nmax
    b = jnp.full((N, 1), beta_ref[0, 0], jnp.float32)
    pad = jnp.zeros((N, TW - D - 4), jnp.float32)
    row = jnp.concatenate([nh, nmax, nmax, b, b, pad], axis=1)
    tbl_ref[...] = row.astype(jnp.bfloat16)
    tbld_ref[...] = nh.astype(jnp.bfloat16)


def _prep(featp, beta):
    return pl.pallas_call(
        _prep_body,
        in_specs=[
            pl.BlockSpec(memory_space=pltpu.SMEM),
            pl.BlockSpec(memory_space=pltpu.VMEM),
        ],
        out_shape=[
            jax.ShapeDtypeStruct((N, TW), jnp.bfloat16),
            jax.ShapeDtypeStruct((N, D), jnp.bfloat16),
        ],
    )(jnp.reshape(beta, (1, 1)), featp)


def _finalize_body(parts_ref, out_ref):
    ext = parts_ref[0] + parts_ref[1]
    den = jnp.maximum(ext[:, D:D + 1], 1e-30)
    out_ref[...] = ext[:, :D] / den


def _finalize(parts):
    return pl.pallas_call(
        _finalize_body,
        out_shape=jax.ShapeDtypeStruct((N, D), jnp.float32),
    )(parts)


def _sc_body(tbl_hbm, tbld_hbm, sd_hbm, out_hbm, acc_sh,
             idx0, idx1, sx0, sx1, fs0, fs1, fd0, fd1, mg0, mg1,
             srow, is0, is1, gs0, gs1, ss0, ss1):
    cid = lax.axis_index("c")
    sid = lax.axis_index("s")
    wid = cid * NS + sid
    idxs = [idx0, idx1]
    sidx = [sx0, sx1]
    fss = [fs0, fs1]
    fds = [fd0, fd1]
    msgs = [mg0, mg1]
    isems = [is0, is1]
    gsems = [gs0, gs1]
    ssems = [ss0, ss1]

    def zrowi(r, _):
        for j in range(EXT // 16):
            mg0[r, pl.ds(j * 16, 16)] = jnp.zeros((16,), jnp.float32)
        return 0
    lax.fori_loop(0, ZR, zrowi, 0)

    def zcopy(k, _):
        pltpu.sync_copy(mg0.at[pl.ds(0, ZR)],
                        acc_sh.at[pl.ds(sid * RPT + k * ZR, ZR)])
        return 0
    lax.fori_loop(0, RPT // ZR, zcopy, 0)

    def zmsg(r, _):
        mg0[r, pl.ds(D, 16)] = jnp.zeros((16,), jnp.float32)
        mg1[r, pl.ds(D, 16)] = jnp.zeros((16,), jnp.float32)
        return 0
    lax.fori_loop(0, C, zmsg, 0)
    plsc.subcore_barrier()

    rows16 = lax.broadcasted_iota(jnp.int32, (16,), 0)
    ebase0 = wid * EPW

    def stage_idx(p, s):
        base = ebase0 + p * C
        pltpu.async_copy(sd_hbm.at[0, pl.ds(base, C)], idxs[s].at[0], isems[s])
        pltpu.async_copy(sd_hbm.at[1, pl.ds(base, C)], idxs[s].at[1], isems[s])

    def issue_gather(p, s):
        base = ebase0 + p * C
        pltpu.make_async_copy(
            sd_hbm.at[0, pl.ds(base, C)], idxs[s].at[0], isems[s]).wait()
        pltpu.make_async_copy(
            sd_hbm.at[1, pl.ds(base, C)], idxs[s].at[1], isems[s]).wait()
        pltpu.async_copy(tbl_hbm.at[idxs[s].at[0]], fss[s], gsems[s])
        pltpu.async_copy(tbld_hbm.at[idxs[s].at[1]], fds[s], gsems[s])

    def wait_gather(s):
        pltpu.make_async_copy(tbl_hbm.at[idxs[s].at[0]], fss[s], gsems[s]).wait()
        pltpu.make_async_copy(tbld_hbm.at[idxs[s].at[1]], fds[s], gsems[s]).wait()

    def issue_scatter(s):
        pltpu.async_copy(msgs[s], acc_sh.at[sidx[s]], ssems[s], add=True)

    def wait_scatter(s):
        pltpu.make_async_copy(msgs[s], acc_sh.at[sidx[s]], ssems[s]).wait()

    def unpack2(v):
        return plsc.unpack(v, format=plsc.PackFormat.INTERLEAVED,
                           preferred_element_type=jnp.float32)

    def compute(p, s):
        fs, fd, msg = fss[s], fds[s], msgs[s]
        bscal = None
        for g in range(GRP):
            e0 = g * 16
            nm = [None] * 16
            for e in range(16):
                row = e0 + e
                dp = None
                for j in range(D // 32):
                    qa, qb = unpack2(fs[row, pl.ds(j * 32, 32)])
                    ta, tb = unpack2(fd[row, pl.ds(j * 32, 32)])
                    term = qa * ta + qb * tb
                    dp = term if dp is None else dp + term
                srow[e, :] = dp
                sa, _sb = unpack2(fs[row, pl.ds(D, 32)])
                nm[e] = sa[0]
                if bscal is None:
                    bscal = sa[1]
            tot = plsc.load_gather(srow, [rows16, jnp.zeros((16,), jnp.int32)])
            for j in range(1, 16):
                tot = tot + plsc.load_gather(
                    srow, [rows16, jnp.full((16,), j, jnp.int32)])
            w = jnp.exp(tot * bscal)
            gidx = ebase0 + p * C + e0 + rows16
            w = jnp.where(gidx < E, w, 0.0)
            for e in range(16):
                row = e0 + e
                ws2 = w[e] * nm[e]
                for j in range(D // 32):
                    qa, qb = unpack2(fs[row, pl.ds(j * 32, 32)])
                    msg[row, pl.ds(j * 32, 16)] = qa * ws2
                    msg[row, pl.ds(j * 32 + 16, 16)] = qb * ws2
            plsc.store_scatter(
                msg, [rows16 + e0, jnp.full((16,), D, jnp.int32)], w)

    stage_idx(0, 0)
    stage_idx(1, 1)
    issue_gather(0, 0)
    issue_gather(1, 1)

    KMAX = NCH // 2

    def body(k, _):
        for r in range(2):
            s = r
            p = 2 * k + r
            wait_gather(s)
            for j in range(C // 16):
                sidx[s][pl.ds(j * 16, 16)] = idxs[s][1, pl.ds(j * 16, 16)]
            @pl.when(k < KMAX - 1)
            def _():
                stage_idx(p + 2, s)
            compute(p, s)
            if r == 0:
                @pl.when(k > 0)
                def _():
                    wait_scatter(1)
            else:
                wait_scatter(0)
            issue_scatter(s)
            @pl.when(k < KMAX - 1)
            def _():
                issue_gather(p + 2, s)
        return 0

    lax.fori_loop(0, KMAX, body, 0)
    wait_scatter(1)
    plsc.subcore_barrier()

    pltpu.sync_copy(acc_sh.at[pl.ds(sid * RPT, RPT)],
                    out_hbm.at[cid, pl.ds(sid * RPT, RPT)])


def _sc_edge_pass(tbl, tbld, sd):
    mesh = plsc.VectorSubcoreMesh(core_axis_name="c", subcore_axis_name="s")
    return pl.kernel(
        _sc_body,
        out_type=jax.ShapeDtypeStruct((NC, N, EXT), jnp.float32),
        mesh=mesh,
        compiler_params=pltpu.CompilerParams(
            use_tc_tiling_on_sc=False, needs_layout_passes=False),
        scratch_types=(
            [pltpu.VMEM_SHARED((N, EXT), jnp.float32)]
            + [pltpu.VMEM((2, C), jnp.int32)] * 2
            + [pltpu.VMEM((C,), jnp.int32)] * 2
            + [pltpu.VMEM((C, TW), jnp.bfloat16)] * 2
            + [pltpu.VMEM((C, D), jnp.bfloat16)] * 2
            + [pltpu.VMEM((C, EXT), jnp.float32)] * 2
            + [pltpu.VMEM((16, 16), jnp.float32)]
            + [pltpu.SemaphoreType.DMA] * 6
        ),
    )(tbl, tbld, sd)


def kernel(feat, edge_index, beta):
    sd = jnp.pad(edge_index.astype(jnp.int32), ((0, 0), (0, EP - E)))
    tbl, tbld = _prep(feat[:, list(_PIN)], beta.astype(jnp.float32))
    parts = _sc_edge_pass(tbl, tbld, sd)
    return _finalize(parts)
